# baseline (speedup 1.0000x reference)
import functools

import jax
import jax.numpy as jnp
from jax.experimental import pallas as pl
from jax.experimental.pallas import tpu as pltpu

TEMP = 0.05
INV_TEMP = 1.0 / TEMP


def _ce_block_kernel(lhs_ref, targets_ref, feat_ref, out_ref,
                     m_ref, s_ref, tl_ref, *, nblocks, bk2, b):
    i = pl.program_id(0)

    @pl.when(i == 0)
    def _init():
        m_ref[...] = jnp.full_like(m_ref, -jnp.inf)
        s_ref[...] = jnp.zeros_like(s_ref)
        tl_ref[...] = jnp.zeros_like(tl_ref)

    lhs = lhs_ref[...]
    f = feat_ref[...]
    z = jax.lax.dot_general(
        lhs, f, (((1,), (1,)), ((), ())),
        preferred_element_type=jnp.float32) * INV_TEMP

    b2 = 2 * b
    cols = (i * (2 * bk2)
            + 2 * jax.lax.broadcasted_iota(jnp.int32, (b2, bk2), 1)
            + (jax.lax.broadcasted_iota(jnp.int32, (b2, bk2), 0) >= b))
    t = targets_ref[...]
    tl_ref[...] += jnp.sum(jnp.where(cols == t, z, 0.0), axis=1,
                           keepdims=True)

    bm = jnp.max(z, axis=1, keepdims=True)
    m_old = m_ref[...]
    m_new = jnp.maximum(m_old, bm)
    s_ref[...] = (s_ref[...] * jnp.exp(m_old - m_new)
                  + jnp.sum(jnp.exp(z - m_new), axis=1, keepdims=True))
    m_ref[...] = m_new

    @pl.when(i == nblocks - 1)
    def _fin():
        m1, m2 = m_ref[:b, :], m_ref[b:, :]
        mm = jnp.maximum(m1, m2)
        s = s_ref[:b, :] * jnp.exp(m1 - mm) + s_ref[b:, :] * jnp.exp(m2 - mm)
        tl = tl_ref[:b, :] + tl_ref[b:, :]
        nll = mm + jnp.log(s) - tl
        out_ref[0, 0] = jnp.mean(nll)


def _pick_block(n):
    for bk in (20000, 10000, 5000, 4000, 2000, 1000, 800, 400, 200, 40, 8):
        if n % bk == 0:
            return bk
    return n


@jax.jit
def kernel(inputs, targets, features):
    b, d = inputs.shape
    m, _ = features.shape
    m2 = m // 2
    f2 = features.reshape(m2, 2 * d)
    bk2 = _pick_block(m2)
    nblocks = m2 // bk2

    zpad = jnp.zeros_like(inputs)
    lhs = jnp.concatenate(
        [jnp.concatenate([inputs, zpad], axis=1),
         jnp.concatenate([zpad, inputs], axis=1)], axis=0)

    t32 = targets.astype(jnp.int32)
    t2d = jnp.concatenate([t32, t32]).reshape(2 * b, 1)

    out = pl.pallas_call(
        functools.partial(_ce_block_kernel, nblocks=nblocks, bk2=bk2, b=b),
        grid=(nblocks,),
        in_specs=[
            pl.BlockSpec((2 * b, 2 * d), lambda i: (0, 0)),
            pl.BlockSpec((2 * b, 1), lambda i: (0, 0)),
            pl.BlockSpec((bk2, 2 * d), lambda i: (i, 0)),
        ],
        out_specs=pl.BlockSpec(memory_space=pltpu.SMEM),
        out_shape=jax.ShapeDtypeStruct((1, 1), jnp.float32),
        scratch_shapes=[
            pltpu.VMEM((2 * b, 1), jnp.float32),
            pltpu.VMEM((2 * b, 1), jnp.float32),
            pltpu.VMEM((2 * b, 1), jnp.float32),
        ],
        compiler_params=pltpu.CompilerParams(
            dimension_semantics=("arbitrary",)),
    )(lhs, t2d, f2)
    return out[0, 0]


# speedup vs baseline: 1.4110x; 1.4110x over previous
import functools
import math

import jax
import jax.numpy as jnp
from jax.experimental import pallas as pl
from jax.experimental.pallas import tpu as pltpu

TEMP = 0.05
LN2 = math.log(2.0)
SCALE = 1.0 / (TEMP * LN2)


def _ce_block_kernel(x_ref, t_ref, feat_ref, fhbm_ref, out_ref,
                     m_ref, s_ref, tf_ref, gsem, *, nblocks, b):
    i = pl.program_id(0)

    @pl.when(i == 0)
    def _init():
        m_ref[...] = jnp.full_like(m_ref, -jnp.inf)
        s_ref[...] = jnp.zeros_like(s_ref)
        for bb in range(b):
            tb = t_ref[bb, 0]
            pltpu.make_async_copy(
                fhbm_ref.at[pl.ds(tb, 1), :],
                tf_ref.at[pl.ds(bb, 1), :],
                gsem).start()

    x = x_ref[...]
    f = feat_ref[...]
    z = jax.lax.dot_general(
        x, f, (((1,), (1,)), ((), ())),
        preferred_element_type=jnp.float32)

    bm = jnp.max(z, axis=1, keepdims=True)
    m_old = m_ref[...]
    m_new = jnp.maximum(m_old, bm)
    s_ref[...] = (s_ref[...] * jnp.exp2(m_old - m_new)
                  + jnp.sum(jnp.exp2(z - m_new), axis=1, keepdims=True))
    m_ref[...] = m_new

    @pl.when(i == nblocks - 1)
    def _fin():
        for bb in range(b):
            pltpu.make_async_copy(
                fhbm_ref.at[pl.ds(0, 1), :],
                tf_ref.at[pl.ds(bb, 1), :],
                gsem).wait()
        tl = jnp.sum(x * tf_ref[...], axis=1, keepdims=True)
        nll = LN2 * (m_ref[...] + jnp.log2(s_ref[...]) - tl)
        out_ref[0, 0] = jnp.mean(nll)


def _pick_block(n):
    for bk in (25000, 20000, 10000, 8000, 5000, 4000, 2000, 1000, 800,
               400, 200, 40, 8):
        if n % bk == 0:
            return bk
    return n


@jax.jit
def kernel(inputs, targets, features):
    b, d = inputs.shape
    m, _ = features.shape
    bk = _pick_block(m)
    nblocks = m // bk

    x = inputs * jnp.float32(SCALE)
    t2d = targets.astype(jnp.int32).reshape(b, 1)

    out = pl.pallas_call(
        functools.partial(_ce_block_kernel, nblocks=nblocks, b=b),
        grid=(nblocks,),
        in_specs=[
            pl.BlockSpec((b, d), lambda i: (0, 0)),
            pl.BlockSpec(memory_space=pltpu.SMEM),
            pl.BlockSpec((bk, d), lambda i: (i, 0)),
            pl.BlockSpec(memory_space=pltpu.MemorySpace.HBM),
        ],
        out_specs=pl.BlockSpec(memory_space=pltpu.SMEM),
        out_shape=jax.ShapeDtypeStruct((1, 1), jnp.float32),
        scratch_shapes=[
            pltpu.VMEM((b, 1), jnp.float32),
            pltpu.VMEM((b, 1), jnp.float32),
            pltpu.VMEM((b, d), jnp.float32),
            pltpu.SemaphoreType.DMA,
        ],
        compiler_params=pltpu.CompilerParams(
            dimension_semantics=("arbitrary",)),
    )(x, t2d, features, features)
    return out[0, 0]


# speedup vs baseline: 1.4196x; 1.0061x over previous
import functools
import math

import jax
import jax.numpy as jnp
from jax.experimental import pallas as pl
from jax.experimental.pallas import tpu as pltpu

TEMP = 0.05
LN2 = math.log(2.0)
SCALE = 1.0 / (TEMP * LN2)
NSTREAM = 4


def _ce_block_kernel(x_ref, t_ref, *rest, nblocks, b):
    feat_refs = rest[:NSTREAM]
    fhbm_ref = rest[NSTREAM]
    out_ref = rest[NSTREAM + 1]
    m_ref, s_ref, tf_ref, gsem = rest[NSTREAM + 2:]

    i = pl.program_id(0)

    @pl.when(i == 0)
    def _init():
        m_ref[...] = jnp.full_like(m_ref, -jnp.inf)
        s_ref[...] = jnp.zeros_like(s_ref)
        for bb in range(b):
            tb = t_ref[bb, 0]
            pltpu.make_async_copy(
                fhbm_ref.at[pl.ds(tb, 1), :],
                tf_ref.at[pl.ds(bb, 1), :],
                gsem).start()

    x = x_ref[...]
    m_old = m_ref[...]
    s_old = s_ref[...]
    for k in range(NSTREAM):
        f = feat_refs[k][...]
        z = jax.lax.dot_general(
            x, f, (((1,), (1,)), ((), ())),
            preferred_element_type=jnp.float32)
        bm = jnp.max(z, axis=1, keepdims=True)
        m_new = jnp.maximum(m_old, bm)
        s_old = (s_old * jnp.exp2(m_old - m_new)
                 + jnp.sum(jnp.exp2(z - m_new), axis=1, keepdims=True))
        m_old = m_new
    m_ref[...] = m_old
    s_ref[...] = s_old

    @pl.when(i == nblocks - 1)
    def _fin():
        for bb in range(b):
            pltpu.make_async_copy(
                fhbm_ref.at[pl.ds(0, 1), :],
                tf_ref.at[pl.ds(bb, 1), :],
                gsem).wait()
        tl = jnp.sum(x * tf_ref[...], axis=1, keepdims=True)
        nll = LN2 * (m_ref[...] + jnp.log2(s_ref[...]) - tl)
        out_ref[0, 0] = jnp.mean(nll)


def _pick_block(n):
    for bk in (10000, 8000, 5000, 4000, 2000, 1000, 800, 400, 200, 40, 8):
        if n % bk == 0:
            return bk
    return n


@jax.jit
def kernel(inputs, targets, features):
    b, d = inputs.shape
    m, _ = features.shape
    bk = _pick_block(m // NSTREAM)
    nblocks = m // (bk * NSTREAM)

    x = inputs * jnp.float32(SCALE)
    t2d = targets.astype(jnp.int32).reshape(b, 1)

    def _mk_spec(k):
        return pl.BlockSpec((bk, d), lambda i, kk=k: (NSTREAM * i + kk, 0))

    out = pl.pallas_call(
        functools.partial(_ce_block_kernel, nblocks=nblocks, b=b),
        grid=(nblocks,),
        in_specs=[
            pl.BlockSpec((b, d), lambda i: (0, 0)),
            pl.BlockSpec(memory_space=pltpu.SMEM),
        ] + [_mk_spec(k) for k in range(NSTREAM)] + [
            pl.BlockSpec(memory_space=pltpu.MemorySpace.HBM),
        ],
        out_specs=pl.BlockSpec(memory_space=pltpu.SMEM),
        out_shape=jax.ShapeDtypeStruct((1, 1), jnp.float32),
        scratch_shapes=[
            pltpu.VMEM((b, 1), jnp.float32),
            pltpu.VMEM((b, 1), jnp.float32),
            pltpu.VMEM((b, d), jnp.float32),
            pltpu.SemaphoreType.DMA,
        ],
        compiler_params=pltpu.CompilerParams(
            dimension_semantics=("arbitrary",)),
    )(x, t2d, *([features] * NSTREAM), features)
    return out[0, 0]
